# Initial kernel scaffold; baseline (speedup 1.0000x reference)
import jax
import jax.numpy as jnp
from jax.experimental import pallas as pl


def kernel(feat_img, lidar2img, w_dn, b_dn):
    raise NotImplementedError("write your pallas kernel here")


# speedup vs baseline: 32.2842x; 32.2842x over previous
import numpy as np

import jax
import jax.numpy as jnp
from jax.experimental import pallas as pl
from jax.experimental.pallas import tpu as pltpu

_IMAGE_SIZE = (256, 704)
_DBOUND = (1.0, 60.0, 0.5)
_XBOUND = (0.0, 72.0, 0.4)
_YBOUND = (-16.0, 16.0, 0.4)
_ZBOUND = (-2.0, 7.6, 9.6)

_D = int((_DBOUND[1] - _DBOUND[0]) / _DBOUND[2])
_DP = 120

_dxg = np.array([b[2] for b in (_XBOUND, _YBOUND, _ZBOUND)], np.float32)
_bxg = np.array([b[0] + b[2] / 2.0 for b in (_XBOUND, _YBOUND, _ZBOUND)], np.float32)
_nxg = np.array([int((b[1] - b[0]) / b[2]) for b in (_XBOUND, _YBOUND, _ZBOUND)], np.int32)
_NX, _NY, _NZ = int(_nxg[0]), int(_nxg[1]), int(_nxg[2])

_BIG = 1000000

_INTERPRET = False


def _cam_kernel(wD_ref, bD_ref, wCT_ref, bC_ref,
                feat_ref, featT_ref, gxlo_ref, gxhi_ref, gy_ref,
                out_ref,
                depth_s, ctxT_s):
    P = feat_ref.shape[2]

    feat = feat_ref[0]
    outD = jnp.dot(wD_ref[...], feat,
                   preferred_element_type=jnp.float32) + bD_ref[...]
    depth_s[...] = jax.nn.softmax(outD, axis=0)
    ctxT_s[...] = jnp.dot(featT_ref[0], wCT_ref[...],
                          preferred_element_type=jnp.float32) + bC_ref[...]

    out_ref[...] = jnp.zeros(out_ref.shape, out_ref.dtype)
    yi = jax.lax.broadcasted_iota(jnp.int32, (_NY, 1), 0)

    def chunk_body(dc, carry):
        base = pl.multiple_of(dc * 8, 8)
        depc = depth_s[pl.ds(base, 8), :]
        gxloc = gxlo_ref[0, pl.ds(base, 8), :]
        gxhic = gxhi_ref[0, pl.ds(base, 8), :]
        gyc = gy_ref[0, pl.ds(base, 8), :]
        for i in range(8):
            dep = depc[i:i + 1, :]
            gxr = gxloc[i:i + 1, :]
            gyr = gyc[i:i + 1, :]
            gmin = jnp.min(gxr)
            gmax = jnp.max(gxhic[i:i + 1, :])
            cmpY = gyr == yi

            def slab_body(g, c2):
                depg = jnp.where(gxr == g, dep, 0.0)
                Wd = jnp.where(cmpY, jnp.broadcast_to(depg, (_NY, P)), 0.0)
                inc = jnp.dot(Wd, ctxT_s[...],
                              preferred_element_type=jnp.float32)
                cur = out_ref[0, pl.ds(g, 1)]
                out_ref[0, pl.ds(g, 1)] = cur + inc[None]
                return c2

            jax.lax.fori_loop(gmin, gmax + 1, slab_body, 0)
        return carry

    jax.lax.fori_loop(0, _DP // 8, chunk_body, 0)


def _bin_indices(lidar2img, fH, fW):
    iH, iW = _IMAGE_SIZE
    ds = jnp.arange(_DBOUND[0], _DBOUND[1], _DBOUND[2], dtype=jnp.float32)
    xs = jnp.linspace(0.0, iW - 1.0, fW, dtype=jnp.float32)
    ys = jnp.linspace(0.0, iH - 1.0, fH, dtype=jnp.float32)
    fr_x = jnp.broadcast_to(xs[None, None, :], (_D, fH, fW))
    fr_y = jnp.broadcast_to(ys[None, :, None], (_D, fH, fW))
    fr_d = jnp.broadcast_to(ds[:, None, None], (_D, fH, fW))
    pts = jnp.stack([fr_x * fr_d, fr_y * fr_d, fr_d, jnp.ones_like(fr_d)], axis=-1)
    inv = jnp.linalg.inv(lidar2img)
    geom = jnp.einsum('bnij,dhwj->bndhwi', inv, pts)[..., :3]
    coords = ((geom - (_bxg - _dxg / 2.0)) / _dxg).astype(jnp.int32)
    gx, gy, gz = coords[..., 0], coords[..., 1], coords[..., 2]
    kept = ((gx >= 0) & (gx < _NX) & (gy >= 0) & (gy < _NY)
            & (gz >= 0) & (gz < _NZ))
    gxlo = jnp.where(kept, gx, _BIG)
    gxhi = jnp.where(kept, gx, -_BIG)
    return gxlo, gxhi, gy


def kernel(feat_img, lidar2img, w_dn, b_dn):
    B, N, C, fH, fW = feat_img.shape
    BN = B * N
    P = fH * fW

    feat2 = feat_img.reshape(BN, C, P)
    featT = feat2.transpose(0, 2, 1)

    gxlo, gxhi, gy = _bin_indices(lidar2img, fH, fW)
    pad = ((0, 0), (0, _DP - _D), (0, 0))
    gxlo = jnp.pad(gxlo.reshape(BN, _D, P), pad, constant_values=_BIG)
    gxhi = jnp.pad(gxhi.reshape(BN, _D, P), pad, constant_values=-_BIG)
    gy = jnp.pad(gy.reshape(BN, _D, P), pad, constant_values=0)

    wD = jnp.concatenate([w_dn[:_D], jnp.zeros((_DP - _D, C), jnp.float32)], axis=0)
    bD = jnp.concatenate([b_dn[:_D], jnp.full((_DP - _D,), -1e30, jnp.float32)]).reshape(_DP, 1)
    wCT = w_dn[_D:_D + C].T
    bC = b_dn[_D:_D + C].reshape(1, C)

    out = pl.pallas_call(
        _cam_kernel,
        out_shape=jax.ShapeDtypeStruct((BN, _NX, _NY, C), jnp.float32),
        grid=(BN,),
        in_specs=[
            pl.BlockSpec((_DP, C), lambda b: (0, 0)),
            pl.BlockSpec((_DP, 1), lambda b: (0, 0)),
            pl.BlockSpec((C, C), lambda b: (0, 0)),
            pl.BlockSpec((1, C), lambda b: (0, 0)),
            pl.BlockSpec((1, C, P), lambda b: (b, 0, 0)),
            pl.BlockSpec((1, P, C), lambda b: (b, 0, 0)),
            pl.BlockSpec((1, _DP, P), lambda b: (b, 0, 0)),
            pl.BlockSpec((1, _DP, P), lambda b: (b, 0, 0)),
            pl.BlockSpec((1, _DP, P), lambda b: (b, 0, 0)),
        ],
        out_specs=pl.BlockSpec((1, _NX, _NY, C), lambda b: (b, 0, 0, 0)),
        scratch_shapes=[
            pltpu.VMEM((_DP, P), jnp.float32),
            pltpu.VMEM((P, C), jnp.float32),
        ],
        compiler_params=pltpu.CompilerParams(
            dimension_semantics=("parallel",),
            vmem_limit_bytes=50 * 1024 * 1024,
        ),
        name="cam_bev_pool",
        interpret=_INTERPRET,
    )(wD, bD, wCT, bC, feat2, featT, gxlo, gxhi, gy)

    return out.transpose(0, 3, 2, 1)
